# baseline (speedup 1.0000x reference)
import jax
import jax.numpy as jnp
from jax import lax
from jax.experimental import pallas as pl
from jax.experimental.pallas import tpu as pltpu

EPS = 1e-05
_BLOCK_ROWS = 2048


def _pdf_block(w_ref, o_ref):
    w = w_ref[...]
    lane = lax.broadcasted_iota(jnp.int32, w.shape, 1)
    m = lane < 64
    s_tot = jnp.sum(w, axis=1, keepdims=True)
    s0 = jnp.sum(jnp.where(m, w, 0.0), axis=1, keepdims=True)
    s1 = s_tot - s0
    s = jnp.where(m, s0, s1)
    pad = jnp.maximum(EPS - s, 0.0)
    o_ref[...] = (w + pad * (1.0 / 64.0)) / (s + pad)


def kernel(weights, stratified):
    n, d = weights.shape
    w2 = jnp.reshape(weights, (n // 2, 2 * d))
    out = pl.pallas_call(
        _pdf_block,
        grid=(w2.shape[0] // _BLOCK_ROWS,),
        in_specs=[pl.BlockSpec((_BLOCK_ROWS, 2 * d), lambda i: (i, 0))],
        out_specs=pl.BlockSpec((_BLOCK_ROWS, 2 * d), lambda i: (i, 0)),
        out_shape=jax.ShapeDtypeStruct(w2.shape, weights.dtype),
        compiler_params=pltpu.CompilerParams(
            dimension_semantics=("parallel",),
        ),
    )(w2)
    return jnp.reshape(out, (n, d))


# speedup vs baseline: 1.2502x; 1.2502x over previous
import jax
import jax.numpy as jnp
from jax import lax
from jax.experimental import pallas as pl
from jax.experimental.pallas import tpu as pltpu

EPS = 1e-05
_BLOCK_ROWS = 1024


def _pdf_block(w_ref, o_ref):
    w = w_ref[...]
    s = jnp.sum(w, axis=1, keepdims=True)
    pad = jnp.maximum(EPS - s, 0.0)
    o_ref[...] = (w + pad * (1.0 / w.shape[1])) / (s + pad)


def kernel(weights, stratified):
    n, d = weights.shape
    return pl.pallas_call(
        _pdf_block,
        grid=(n // _BLOCK_ROWS,),
        in_specs=[pl.BlockSpec((_BLOCK_ROWS, d), lambda i: (i, 0))],
        out_specs=pl.BlockSpec((_BLOCK_ROWS, d), lambda i: (i, 0)),
        out_shape=jax.ShapeDtypeStruct((n, d), weights.dtype),
        compiler_params=pltpu.CompilerParams(
            dimension_semantics=("parallel",),
        ),
    )(weights)


# speedup vs baseline: 9.0969x; 7.2762x over previous
import jax
import jax.numpy as jnp
from jax.experimental import pallas as pl
from jax.experimental.pallas import tpu as pltpu

EPS = 1e-05
_BLOCK_COLS = 8192


def _pdf_block(w_ref, o_ref):
    w = w_ref[...]
    s = jnp.sum(w, axis=0, keepdims=True)
    pad = jnp.maximum(EPS - s, 0.0)
    inv = 1.0 / (s + pad)
    o_ref[...] = (w + pad * (1.0 / w.shape[0])) * inv


def kernel(weights, stratified):
    n, d = weights.shape
    wt = weights.T
    out_t = pl.pallas_call(
        _pdf_block,
        grid=(n // _BLOCK_COLS,),
        in_specs=[pl.BlockSpec((d, _BLOCK_COLS), lambda i: (0, i))],
        out_specs=pl.BlockSpec((d, _BLOCK_COLS), lambda i: (0, i)),
        out_shape=jax.ShapeDtypeStruct((d, n), weights.dtype),
        compiler_params=pltpu.CompilerParams(
            dimension_semantics=("parallel",),
        ),
    )(wt)
    return out_t.T
